# Initial kernel scaffold; baseline (speedup 1.0000x reference)
import functools

import jax
import jax.numpy as jnp
from jax import lax
from jax.experimental import pallas as pl
from jax.experimental.pallas import tpu as pltpu, tpu_sc as plsc

E = 1_600_000
D = 16
OUT_D = 46
L = 16
B = 512
NW = 32
N_CHUNKS = E // B
CHUNKS_PER_W = -(-N_CHUNKS // NW)
UNROLL = 8


def _sc_body(ea_hbm, wt_hbm, wr_hbm, out_hbm,
             wt_v, wr_v, ea_v, out_v, idx_t_v, idx_r_v):
    wid = lax.axis_index("s") * 2 + lax.axis_index("c")
    pltpu.sync_copy(wt_hbm, wt_v)
    pltpu.sync_copy(wr_hbm, wr_v)

    lanes = lax.iota(jnp.int32, (L,))
    zeros = jnp.zeros((L,), jnp.int32)
    ones = jnp.ones((L,), jnp.int32)

    def chunk_body(k, carry):
        g = wid + k * NW

        @pl.when(g < N_CHUNKS)
        def _():
            base = g * B
            pltpu.sync_copy(ea_hbm.at[pl.ds(base, B)], ea_v)

            def grp(i, c):
                rows = i * L + lanes
                tvec = plsc.load_gather(ea_v, [rows, zeros])
                rvec = plsc.load_gather(ea_v, [rows, ones])
                idx_t_v[pl.ds(i * L, L)] = tvec.astype(jnp.int32)
                idx_r_v[pl.ds(i * L, L)] = rvec.astype(jnp.int32)
                return c

            lax.fori_loop(0, B // L, grp, 0)

            def edge(j, c):
                for u in range(UNROLL):
                    e = j * UNROLL + u
                    t = idx_t_v[e]
                    r = idx_r_v[e]
                    out_v[e, pl.ds(30, L)] = ea_v[e, :]
                    out_v[e, pl.ds(16, L)] = wr_v[r, :]
                    out_v[e, pl.ds(0, L)] = wt_v[t, :]
                return c

            lax.fori_loop(0, B // UNROLL, edge, 0)

            pltpu.sync_copy(out_v, out_hbm.at[pl.ds(base, B)])

        return carry

    lax.fori_loop(0, CHUNKS_PER_W, chunk_body, 0)


@jax.jit
def _run(edge_attr, W_type, W_ring):
    mesh = plsc.VectorSubcoreMesh(core_axis_name="c", subcore_axis_name="s")
    f = pl.kernel(
        _sc_body,
        out_type=jax.ShapeDtypeStruct((E, OUT_D), jnp.float32),
        mesh=mesh,
        scratch_types=[
            pltpu.VMEM((10, D), jnp.float32),
            pltpu.VMEM((10, D), jnp.float32),
            pltpu.VMEM((B, D), jnp.float32),
            pltpu.VMEM((B, OUT_D), jnp.float32),
            pltpu.VMEM((B,), jnp.int32),
            pltpu.VMEM((B,), jnp.int32),
        ],
    )
    return f(edge_attr, W_type, W_ring)


def kernel(edge_attr, W_type, W_ring):
    return _run(edge_attr, W_type, W_ring)


# speedup vs baseline: 3.4354x; 3.4354x over previous
import jax
import jax.numpy as jnp
from jax import lax
from jax.experimental import pallas as pl
from jax.experimental.pallas import tpu as pltpu, tpu_sc as plsc

E = 1_600_000
D = 16
OUT_D = 46
L = 16
B = 512
NW = 32
N_CHUNKS = E // B
CHUNKS_PER_W = -(-N_CHUNKS // NW)


def _sc_body(ea_hbm, wt_hbm, wr_hbm, out_hbm, wt_v, wr_v, ea_v, out_v):
    wid = lax.axis_index("s") * 2 + lax.axis_index("c")
    pltpu.sync_copy(wt_hbm, wt_v)
    pltpu.sync_copy(wr_hbm, wr_v)

    def chunk_body(k, carry):
        g = wid + k * NW

        @pl.when(g < N_CHUNKS)
        def _():
            base = g * B
            pltpu.sync_copy(ea_hbm.at[pl.ds(base * D, B * D)], ea_v)

            def grp(i, c):
                for u in range(L):
                    e = i * L + u
                    feat = ea_v[pl.ds(e * D, L)]
                    fi = feat.astype(jnp.int32)
                    t = fi[0]
                    r = fi[1]
                    out_v[e, pl.ds(30, L)] = feat
                    out_v[e, pl.ds(16, L)] = wr_v[pl.ds(r * D, L)]
                    out_v[e, pl.ds(0, L)] = wt_v[pl.ds(t * D, L)]
                return c

            lax.fori_loop(0, B // L, grp, 0)

            pltpu.sync_copy(out_v, out_hbm.at[pl.ds(base, B)])

        return carry

    lax.fori_loop(0, CHUNKS_PER_W, chunk_body, 0)


@jax.jit
def _run(ea_flat, wt_flat, wr_flat):
    mesh = plsc.VectorSubcoreMesh(core_axis_name="c", subcore_axis_name="s")
    f = pl.kernel(
        _sc_body,
        out_type=jax.ShapeDtypeStruct((E, OUT_D), jnp.float32),
        mesh=mesh,
        scratch_types=[
            pltpu.VMEM((10 * D,), jnp.float32),
            pltpu.VMEM((10 * D,), jnp.float32),
            pltpu.VMEM((B * D,), jnp.float32),
            pltpu.VMEM((B, OUT_D), jnp.float32),
        ],
    )
    return f(ea_flat, wt_flat, wr_flat)


def kernel(edge_attr, W_type, W_ring):
    return _run(edge_attr.reshape(E * D),
                W_type.reshape(10 * D),
                W_ring.reshape(10 * D))


# speedup vs baseline: 5.1523x; 1.4998x over previous
import jax
import jax.numpy as jnp
from jax import lax
from jax.experimental import pallas as pl
from jax.experimental.pallas import tpu as pltpu, tpu_sc as plsc

E = 1_600_000
D = 16
OUT_D = 46
L = 16
B = 512
NW = 32
N_CHUNKS = E // B
CHUNKS_PER_W = -(-N_CHUNKS // NW)


def _sc_body(ea_hbm, wt_hbm, wr_hbm, out_hbm, wt_v, wr_v, ea_v, out_v):
    wid = lax.axis_index("s") * 2 + lax.axis_index("c")
    pltpu.sync_copy(wt_hbm, wt_v)
    pltpu.sync_copy(wr_hbm, wr_v)

    def chunk_body(k, carry):
        g = wid + k * NW

        @pl.when(g < N_CHUNKS)
        def _():
            base = g * B
            pltpu.sync_copy(ea_hbm.at[pl.ds(base * D, B * D)], ea_v)

            @plsc.parallel_loop(0, B, step=L, unroll=2)
            def grp(i):
                for u in range(L):
                    e = i + u
                    feat = ea_v[pl.ds(e * D, L)]
                    fi = feat.astype(jnp.int32)
                    t = fi[0]
                    r = fi[1]
                    out_v[e, pl.ds(30, L)] = feat
                    out_v[e, pl.ds(16, L)] = wr_v[pl.ds(r * D, L)]
                    out_v[e, pl.ds(0, L)] = wt_v[pl.ds(t * D, L)]

            pltpu.sync_copy(out_v, out_hbm.at[pl.ds(base, B)])

        return carry

    lax.fori_loop(0, CHUNKS_PER_W, chunk_body, 0)


@jax.jit
def _run(ea_flat, wt_flat, wr_flat):
    mesh = plsc.VectorSubcoreMesh(core_axis_name="c", subcore_axis_name="s")
    f = pl.kernel(
        _sc_body,
        out_type=jax.ShapeDtypeStruct((E, OUT_D), jnp.float32),
        mesh=mesh,
        scratch_types=[
            pltpu.VMEM((10 * D,), jnp.float32),
            pltpu.VMEM((10 * D,), jnp.float32),
            pltpu.VMEM((B * D,), jnp.float32),
            pltpu.VMEM((B, OUT_D), jnp.float32),
        ],
    )
    return f(ea_flat, wt_flat, wr_flat)


def kernel(edge_attr, W_type, W_ring):
    return _run(edge_attr.reshape(E * D),
                W_type.reshape(10 * D),
                W_ring.reshape(10 * D))
